# baseline (speedup 1.0000x reference)
import functools

import jax
import jax.numpy as jnp
from jax import lax
from jax.experimental import pallas as pl
from jax.experimental.pallas import tpu as pltpu
from jax.experimental.pallas import tpu_sc as plsc

_NC = 2
_NS = 16
_NW = _NC * _NS


@functools.lru_cache(maxsize=None)
def _make_gather(V: int, D: int, batch: int, seq: int):
    B = batch * seq
    b_per_w = B // _NW
    chunk = 64
    nbuf = 2
    lookahead = 1
    n_chunks = b_per_w // chunk
    assert seq % b_per_w == 0
    mesh = plsc.VectorSubcoreMesh(core_axis_name="c", subcore_axis_name="s")

    @functools.partial(
        pl.kernel,
        mesh=mesh,
        out_type=jax.ShapeDtypeStruct((batch, seq, D), jnp.float32),
        scratch_types=(
            [pltpu.VMEM((b_per_w,), jnp.int32)]
            + [pltpu.VMEM((chunk, D), jnp.float32) for _ in range(nbuf)]
            + [pltpu.SemaphoreType.DMA for _ in range(2 * nbuf)]
        ),
    )
    def gather_kernel(ids_hbm, table_hbm, out_hbm, idx_v, *rest):
        bufs = rest[:nbuf]
        gsems = rest[nbuf:2 * nbuf]
        wsems = rest[2 * nbuf:3 * nbuf]
        wid = lax.axis_index("s") * _NC + lax.axis_index("c")
        row = wid // (seq // b_per_w)
        off = (wid % (seq // b_per_w)) * b_per_w
        pltpu.sync_copy(ids_hbm.at[row, pl.ds(off, b_per_w)], idx_v)
        gcp = [None] * n_chunks
        wcp = [None] * n_chunks

        def issue_gather(ch):
            b = ch % nbuf
            gcp[ch] = pltpu.async_copy(
                table_hbm.at[idx_v.at[pl.ds(ch * chunk, chunk)]],
                bufs[b], gsems[b])

        for ch in range(min(lookahead, n_chunks)):
            issue_gather(ch)
        for ch in range(n_chunks):
            b = ch % nbuf
            gcp[ch].wait()
            wcp[ch] = pltpu.async_copy(
                bufs[b], out_hbm.at[row, pl.ds(off + ch * chunk, chunk)],
                wsems[b])
            pre = ch + lookahead
            if pre < n_chunks:
                if pre - nbuf >= 0:
                    wcp[pre - nbuf].wait()
                issue_gather(pre)
        for ch in range(max(0, n_chunks - nbuf), n_chunks):
            wcp[ch].wait()

    return gather_kernel


def kernel(input_ids, token_embed):
    batch, seq = input_ids.shape
    vocab, d_model = token_embed.shape
    ids = input_ids.astype(jnp.int32)
    return _make_gather(vocab, d_model, batch, seq)(ids, token_embed)


# speedup vs baseline: 1.0196x; 1.0196x over previous
import functools

import jax
import jax.numpy as jnp
from jax import lax
from jax.experimental import pallas as pl
from jax.experimental.pallas import tpu as pltpu
from jax.experimental.pallas import tpu_sc as plsc

_NC = 2
_NS = 16
_NW = _NC * _NS


@functools.lru_cache(maxsize=None)
def _make_gather(V: int, D: int, batch: int, seq: int):
    B = batch * seq
    b_per_w = B // _NW
    chunk = 16
    nbuf = 8
    lookahead = 2
    n_chunks = b_per_w // chunk
    assert seq % b_per_w == 0
    mesh = plsc.VectorSubcoreMesh(core_axis_name="c", subcore_axis_name="s")

    @functools.partial(
        pl.kernel,
        mesh=mesh,
        out_type=jax.ShapeDtypeStruct((batch, seq, D), jnp.float32),
        scratch_types=(
            [pltpu.VMEM((b_per_w,), jnp.int32)]
            + [pltpu.VMEM((chunk, D), jnp.float32) for _ in range(nbuf)]
            + [pltpu.SemaphoreType.DMA for _ in range(2 * nbuf)]
        ),
    )
    def gather_kernel(ids_hbm, table_hbm, out_hbm, idx_v, *rest):
        bufs = rest[:nbuf]
        gsems = rest[nbuf:2 * nbuf]
        wsems = rest[2 * nbuf:3 * nbuf]
        wid = lax.axis_index("s") * _NC + lax.axis_index("c")
        row = wid // (seq // b_per_w)
        off = (wid % (seq // b_per_w)) * b_per_w
        pltpu.sync_copy(ids_hbm.at[row, pl.ds(off, b_per_w)], idx_v)
        gcp = [None] * n_chunks
        wcp = [None] * n_chunks

        def issue_gather(ch):
            b = ch % nbuf
            gcp[ch] = pltpu.async_copy(
                table_hbm.at[idx_v.at[pl.ds(ch * chunk, chunk)]],
                bufs[b], gsems[b])

        for ch in range(min(lookahead, n_chunks)):
            issue_gather(ch)
        for ch in range(n_chunks):
            b = ch % nbuf
            gcp[ch].wait()
            wcp[ch] = pltpu.async_copy(
                bufs[b], out_hbm.at[row, pl.ds(off + ch * chunk, chunk)],
                wsems[b])
            pre = ch + lookahead
            if pre < n_chunks:
                if pre - nbuf >= 0:
                    wcp[pre - nbuf].wait()
                issue_gather(pre)
        for ch in range(max(0, n_chunks - nbuf), n_chunks):
            wcp[ch].wait()

    return gather_kernel


def kernel(input_ids, token_embed):
    batch, seq = input_ids.shape
    vocab, d_model = token_embed.shape
    ids = input_ids.astype(jnp.int32)
    return _make_gather(vocab, d_model, batch, seq)(ids, token_embed)


# speedup vs baseline: 1.0495x; 1.0294x over previous
import functools

import jax
import jax.numpy as jnp
from jax import lax
from jax.experimental import pallas as pl
from jax.experimental.pallas import tpu as pltpu
from jax.experimental.pallas import tpu_sc as plsc

_NC = 2
_NS = 16
_NW = _NC * _NS


@functools.lru_cache(maxsize=None)
def _make_gather(V: int, D: int, batch: int, seq: int):
    B = batch * seq
    b_per_w = B // _NW
    chunk = 32
    nbuf = 4
    lookahead = 3
    n_chunks = b_per_w // chunk
    assert seq % b_per_w == 0
    mesh = plsc.VectorSubcoreMesh(core_axis_name="c", subcore_axis_name="s")

    @functools.partial(
        pl.kernel,
        mesh=mesh,
        out_type=jax.ShapeDtypeStruct((batch, seq, D), jnp.float32),
        scratch_types=(
            [pltpu.VMEM((b_per_w,), jnp.int32)]
            + [pltpu.VMEM((chunk, D), jnp.float32) for _ in range(nbuf)]
            + [pltpu.SemaphoreType.DMA for _ in range(2 * nbuf)]
        ),
    )
    def gather_kernel(ids_hbm, table_hbm, out_hbm, idx_v, *rest):
        bufs = rest[:nbuf]
        gsems = rest[nbuf:2 * nbuf]
        wsems = rest[2 * nbuf:3 * nbuf]
        wid = lax.axis_index("s") * _NC + lax.axis_index("c")
        row = wid // (seq // b_per_w)
        off = (wid % (seq // b_per_w)) * b_per_w
        pltpu.sync_copy(ids_hbm.at[row, pl.ds(off, b_per_w)], idx_v)
        gcp = [None] * n_chunks
        wcp = [None] * n_chunks

        def issue_gather(ch):
            b = ch % nbuf
            gcp[ch] = pltpu.async_copy(
                table_hbm.at[idx_v.at[pl.ds(ch * chunk, chunk)]],
                bufs[b], gsems[b])

        for ch in range(min(lookahead, n_chunks)):
            issue_gather(ch)
        for ch in range(n_chunks):
            b = ch % nbuf
            gcp[ch].wait()
            wcp[ch] = pltpu.async_copy(
                bufs[b], out_hbm.at[row, pl.ds(off + ch * chunk, chunk)],
                wsems[b])
            pre = ch + lookahead
            if pre < n_chunks:
                if pre - nbuf >= 0:
                    wcp[pre - nbuf].wait()
                issue_gather(pre)
        for ch in range(max(0, n_chunks - nbuf), n_chunks):
            wcp[ch].wait()

    return gather_kernel


def kernel(input_ids, token_embed):
    batch, seq = input_ids.shape
    vocab, d_model = token_embed.shape
    ids = input_ids.astype(jnp.int32)
    return _make_gather(vocab, d_model, batch, seq)(ids, token_embed)


# speedup vs baseline: 1.0611x; 1.0111x over previous
import functools

import jax
import jax.numpy as jnp
from jax import lax
from jax.experimental import pallas as pl
from jax.experimental.pallas import tpu as pltpu
from jax.experimental.pallas import tpu_sc as plsc

_NC = 2
_NS = 16
_NW = _NC * _NS


@functools.lru_cache(maxsize=None)
def _make_gather(V: int, D: int, batch: int, seq: int):
    B = batch * seq
    b_per_w = B // _NW
    chunk = 32
    nbuf = 5
    lookahead = 3
    n_chunks = b_per_w // chunk
    assert seq % b_per_w == 0
    mesh = plsc.VectorSubcoreMesh(core_axis_name="c", subcore_axis_name="s")

    @functools.partial(
        pl.kernel,
        mesh=mesh,
        out_type=jax.ShapeDtypeStruct((batch, seq, D), jnp.float32),
        scratch_types=(
            [pltpu.VMEM((b_per_w,), jnp.int32)]
            + [pltpu.VMEM((chunk, D), jnp.float32) for _ in range(nbuf)]
            + [pltpu.SemaphoreType.DMA for _ in range(2 * nbuf)]
        ),
    )
    def gather_kernel(ids_hbm, table_hbm, out_hbm, idx_v, *rest):
        bufs = rest[:nbuf]
        gsems = rest[nbuf:2 * nbuf]
        wsems = rest[2 * nbuf:3 * nbuf]
        wid = lax.axis_index("s") * _NC + lax.axis_index("c")
        row = wid // (seq // b_per_w)
        off = (wid % (seq // b_per_w)) * b_per_w
        pltpu.sync_copy(ids_hbm.at[row, pl.ds(off, b_per_w)], idx_v)
        gcp = [None] * n_chunks
        wcp = [None] * n_chunks

        def issue_gather(ch):
            b = ch % nbuf
            gcp[ch] = pltpu.async_copy(
                table_hbm.at[idx_v.at[pl.ds(ch * chunk, chunk)]],
                bufs[b], gsems[b])

        for ch in range(min(lookahead, n_chunks)):
            issue_gather(ch)
        for ch in range(n_chunks):
            b = ch % nbuf
            gcp[ch].wait()
            wcp[ch] = pltpu.async_copy(
                bufs[b], out_hbm.at[row, pl.ds(off + ch * chunk, chunk)],
                wsems[b])
            pre = ch + lookahead
            if pre < n_chunks:
                if pre - nbuf >= 0:
                    wcp[pre - nbuf].wait()
                issue_gather(pre)
        for ch in range(max(0, n_chunks - nbuf), n_chunks):
            wcp[ch].wait()

    return gather_kernel


def kernel(input_ids, token_embed):
    batch, seq = input_ids.shape
    vocab, d_model = token_embed.shape
    ids = input_ids.astype(jnp.int32)
    return _make_gather(vocab, d_model, batch, seq)(ids, token_embed)


# speedup vs baseline: 1.0657x; 1.0043x over previous
import functools

import jax
import jax.numpy as jnp
from jax import lax
from jax.experimental import pallas as pl
from jax.experimental.pallas import tpu as pltpu
from jax.experimental.pallas import tpu_sc as plsc

_NC = 2
_NS = 16
_NW = _NC * _NS


@functools.lru_cache(maxsize=None)
def _make_gather(V: int, D: int, batch: int, seq: int):
    B = batch * seq
    b_per_w = B // _NW
    chunk = 32
    nbuf = 5
    lookahead = 4
    n_chunks = b_per_w // chunk
    assert seq % b_per_w == 0
    mesh = plsc.VectorSubcoreMesh(core_axis_name="c", subcore_axis_name="s")

    @functools.partial(
        pl.kernel,
        mesh=mesh,
        out_type=jax.ShapeDtypeStruct((batch, seq, D), jnp.float32),
        scratch_types=(
            [pltpu.VMEM((b_per_w,), jnp.int32)]
            + [pltpu.VMEM((chunk, D), jnp.float32) for _ in range(nbuf)]
            + [pltpu.SemaphoreType.DMA for _ in range(2 * nbuf)]
        ),
    )
    def gather_kernel(ids_hbm, table_hbm, out_hbm, idx_v, *rest):
        bufs = rest[:nbuf]
        gsems = rest[nbuf:2 * nbuf]
        wsems = rest[2 * nbuf:3 * nbuf]
        wid = lax.axis_index("s") * _NC + lax.axis_index("c")
        row = wid // (seq // b_per_w)
        off = (wid % (seq // b_per_w)) * b_per_w
        pltpu.sync_copy(ids_hbm.at[row, pl.ds(off, b_per_w)], idx_v)
        gcp = [None] * n_chunks
        wcp = [None] * n_chunks

        def issue_gather(ch):
            b = ch % nbuf
            gcp[ch] = pltpu.async_copy(
                table_hbm.at[idx_v.at[pl.ds(ch * chunk, chunk)]],
                bufs[b], gsems[b])

        for ch in range(min(lookahead, n_chunks)):
            issue_gather(ch)
        for ch in range(n_chunks):
            b = ch % nbuf
            gcp[ch].wait()
            wcp[ch] = pltpu.async_copy(
                bufs[b], out_hbm.at[row, pl.ds(off + ch * chunk, chunk)],
                wsems[b])
            pre = ch + lookahead
            if pre < n_chunks:
                if pre - nbuf >= 0:
                    wcp[pre - nbuf].wait()
                issue_gather(pre)
        for ch in range(max(0, n_chunks - nbuf), n_chunks):
            wcp[ch].wait()

    return gather_kernel


def kernel(input_ids, token_embed):
    batch, seq = input_ids.shape
    vocab, d_model = token_embed.shape
    ids = input_ids.astype(jnp.int32)
    return _make_gather(vocab, d_model, batch, seq)(ids, token_embed)
